# baseline (speedup 1.0000x reference)
import functools

import jax
import jax.numpy as jnp
from jax import lax
from jax.experimental import pallas as pl
from jax.experimental.pallas import tpu as pltpu
from jax.experimental.pallas import tpu_sc as plsc

B, L, D = 4, 4096, 2048
NW = 32
ROWS = B * L
RPW = ROWS // NW
R = 8
CHUNK = R * D
NCHUNK = RPW // R

_mesh = plsc.VectorSubcoreMesh(core_axis_name="c", subcore_axis_name="s")

_vmem = lambda: pltpu.VMEM((CHUNK,), jnp.float32)


@functools.partial(
    pl.kernel,
    out_type=jax.ShapeDtypeStruct((ROWS * D,), jnp.float32),
    mesh=_mesh,
    scratch_types=[
        _vmem(), _vmem(), _vmem(),
        _vmem(), _vmem(), _vmem(),
        pltpu.SemaphoreType.DMA, pltpu.SemaphoreType.DMA,
        pltpu.SemaphoreType.DMA, pltpu.SemaphoreType.DMA,
        pltpu.SemaphoreType.DMA, pltpu.SemaphoreType.DMA,
    ],
)
def _sc_add(x_hbm, emb_hbm, out_hbm, xv0, ev0, ov0, xv1, ev1, ov1,
            sx0, se0, so0, sx1, se1, so1):
    wid = lax.axis_index("s") * 2 + lax.axis_index("c")
    base = wid * RPW
    ebase = lax.rem(base, L)
    bufs = ((xv0, ev0, ov0, sx0, se0, so0), (xv1, ev1, ov1, sx1, se1, so1))

    def xoff(i):
        return (base + i * R) * D

    def eoff(i):
        return (ebase + i * R) * D

    for b in range(2):
        xv, ev, _, sx, se, _ = bufs[b]
        pltpu.async_copy(x_hbm.at[pl.ds(xoff(b), CHUNK)], xv, sx)
        pltpu.async_copy(emb_hbm.at[pl.ds(eoff(b), CHUNK)], ev, se)

    def round_body(k, _):
        for b in range(2):
            i = 2 * k + b
            xv, ev, ov, sx, se, so = bufs[b]
            pltpu.make_async_copy(x_hbm.at[pl.ds(xoff(i), CHUNK)], xv, sx).wait()
            pltpu.make_async_copy(emb_hbm.at[pl.ds(eoff(i), CHUNK)], ev, se).wait()
            @pl.when(k > 0)
            def _wait_store():
                pltpu.make_async_copy(
                    ov, out_hbm.at[pl.ds(xoff(i - 2), CHUNK)], so
                ).wait()

            def vec_body(j, _):
                o = j * 16
                ov[pl.ds(o, 16)] = xv[pl.ds(o, 16)] + ev[pl.ds(o, 16)]
                return 0

            lax.fori_loop(0, CHUNK // 16, vec_body, 0, unroll=8)
            pltpu.async_copy(ov, out_hbm.at[pl.ds(xoff(i), CHUNK)], so)

            @pl.when(i + 2 < NCHUNK)
            def _prefetch():
                pltpu.async_copy(x_hbm.at[pl.ds(xoff(i + 2), CHUNK)], xv, sx)
                pltpu.async_copy(emb_hbm.at[pl.ds(eoff(i + 2), CHUNK)], ev, se)

        return 0

    lax.fori_loop(0, NCHUNK // 2, round_body, 0)
    for b in range(2):
        _, _, ov, _, _, so = bufs[b]
        i = NCHUNK - 2 + b
        pltpu.make_async_copy(ov, out_hbm.at[pl.ds(xoff(i), CHUNK)], so).wait()


def kernel(x, emb_weight):
    out = _sc_add(x.reshape(ROWS * D), emb_weight.reshape(L * D))
    return out.reshape(B, L, D)


# speedup vs baseline: 6.7187x; 6.7187x over previous
import jax
import jax.numpy as jnp
from jax.experimental import pallas as pl

B, L, D = 4, 4096, 2048
BB = 1
BL = 1024


def _add_kernel(x_ref, emb_ref, o_ref):
    o_ref[...] = x_ref[...] + emb_ref[...][None, :, :]


def kernel(x, emb_weight):
    nl = L // BL
    nb = B // BB
    return pl.pallas_call(
        _add_kernel,
        grid=(nl, nb),
        in_specs=[
            pl.BlockSpec((BB, BL, D), lambda l, b: (b, l, 0)),
            pl.BlockSpec((BL, D), lambda l, b: (l, 0)),
        ],
        out_specs=pl.BlockSpec((BB, BL, D), lambda l, b: (b, l, 0)),
        out_shape=jax.ShapeDtypeStruct((B, L, D), x.dtype),
    )(x, emb_weight)
